# baseline (speedup 1.0000x reference)
import functools

import jax
import jax.numpy as jnp
from jax import lax
from jax.experimental import pallas as pl
from jax.experimental.pallas import tpu as pltpu
from jax.experimental.pallas import tpu_sc as plsc

LANES = 16


def kernel(input_ids, token_table, pos_table):
    B, S = input_ids.shape
    V, D = token_table.shape
    NW = 32
    CH = 16
    RW = (B * S) // NW
    T = RW // CH

    idx_r = input_ids.astype(jnp.int32).reshape(NW, 1, RW)

    mesh = plsc.VectorSubcoreMesh(core_axis_name="c", subcore_axis_name="s")

    @functools.partial(
        pl.kernel,
        mesh=mesh,
        out_type=jax.ShapeDtypeStruct((B * S, D), jnp.float32),
        scratch_types=[
            pltpu.VMEM((S, D), jnp.float32),
            pltpu.VMEM((1, RW), jnp.int32),
            pltpu.VMEM((CH, D), jnp.float32),
            pltpu.SemaphoreType.DMA,
        ],
    )
    def k(idx_hbm, tok_hbm, pos_hbm, out_hbm, pos_v, idx_v, buf, sem):
        c = lax.axis_index("c")
        s = lax.axis_index("s")
        wid = s * 2 + c
        pltpu.sync_copy(pos_hbm, pos_v)
        pltpu.sync_copy(idx_hbm.at[wid], idx_v)
        row0 = wid * RW

        def chunk(t, carry):
            pltpu.async_copy(
                tok_hbm.at[idx_v.at[0, pl.ds(t * CH, CH)]], buf, sem
            ).wait()
            base = lax.rem(row0 + t * CH, S)
            for r in range(CH):
                prow = base + r
                prow = lax.select(prow >= S, prow - S, prow)

                def add_j(j, carry2, r=r, prow=prow):
                    sl = pl.ds(j * LANES, LANES)
                    buf[r, sl] += pos_v[prow, sl]
                    return carry2

                lax.fori_loop(0, D // LANES, add_j, 0, unroll=8)
            pltpu.sync_copy(buf, out_hbm.at[pl.ds(row0 + t * CH, CH), :])
            return carry

        lax.fori_loop(0, T, chunk, 0)

    out = k(idx_r, token_table, pos_table)
    return out.reshape(B, S, D)


# speedup vs baseline: 1.4881x; 1.4881x over previous
import functools

import jax
import jax.numpy as jnp
from jax import lax
from jax.experimental import pallas as pl
from jax.experimental.pallas import tpu as pltpu
from jax.experimental.pallas import tpu_sc as plsc

LANES = 16


def kernel(input_ids, token_table, pos_table):
    B, S = input_ids.shape
    V, D = token_table.shape
    NW = 32
    CH = 8
    RW = (B * S) // NW
    T = RW // CH
    NR = T // 4

    idx_r = input_ids.astype(jnp.int32).reshape(NW, 1, RW)

    mesh = plsc.VectorSubcoreMesh(core_axis_name="c", subcore_axis_name="s")

    @functools.partial(
        pl.kernel,
        mesh=mesh,
        out_type=jax.ShapeDtypeStruct((B * S, D), jnp.float32),
        scratch_types=[
            pltpu.VMEM((S, D), jnp.float32),
            pltpu.VMEM((1, RW), jnp.int32),
            pltpu.VMEM((CH, D), jnp.float32),
            pltpu.VMEM((CH, D), jnp.float32),
            pltpu.VMEM((CH, D), jnp.float32),
            pltpu.VMEM((CH, D), jnp.float32),
            pltpu.SemaphoreType.DMA,
            pltpu.SemaphoreType.DMA,
            pltpu.SemaphoreType.DMA,
            pltpu.SemaphoreType.DMA,
            pltpu.SemaphoreType.DMA,
            pltpu.SemaphoreType.DMA,
            pltpu.SemaphoreType.DMA,
            pltpu.SemaphoreType.DMA,
        ],
    )
    def k(idx_hbm, tok_hbm, pos_hbm, out_hbm, pos_v, idx_v,
          b0, b1, b2, b3, g0, g1, g2, g3, s0, s1, s2, s3):
        bufs = [b0, b1, b2, b3]
        gsems = [g0, g1, g2, g3]
        ssems = [s0, s1, s2, s3]
        c = lax.axis_index("c")
        s = lax.axis_index("s")
        wid = s * 2 + c
        row0 = wid * RW
        pltpu.sync_copy(pos_hbm, pos_v)
        pltpu.sync_copy(idx_hbm.at[wid], idx_v)

        def gather(u, slot):
            pltpu.async_copy(
                tok_hbm.at[idx_v.at[0, pl.ds(u * CH, CH)]],
                bufs[slot], gsems[slot])

        def gather_wait(u, slot):
            pltpu.make_async_copy(
                tok_hbm.at[idx_v.at[0, pl.ds(u * CH, CH)]],
                bufs[slot], gsems[slot]).wait()

        def scatter(u, slot):
            pltpu.async_copy(
                bufs[slot], out_hbm.at[pl.ds(row0 + u * CH, CH), :],
                ssems[slot])

        def scatter_wait(u, slot):
            pltpu.make_async_copy(
                bufs[slot], out_hbm.at[pl.ds(row0 + u * CH, CH), :],
                ssems[slot]).wait()

        gather(0, 0)
        gather(1, 1)

        def round_fn(rd, carry):
            for j in range(4):
                u = rd * 4 + j
                pslot = (j - 2) % 4
                if j < 2:
                    @pl.when(rd > 0)
                    def _():
                        scatter_wait(u - 2, pslot)
                else:
                    scatter_wait(u - 2, pslot)
                if j < 2:
                    gather(u + 2, pslot)
                else:
                    @pl.when(rd < NR - 1)
                    def _():
                        gather(u + 2, pslot)
                gather_wait(u, j)
                base = lax.rem(row0 + u * CH, S)

                def row_fn(r, carry2, slot=j):
                    prow = base + r
                    prow = lax.select(prow >= S, prow - S, prow)

                    def add_j(jj, carry3):
                        sl = pl.ds(jj * LANES, LANES)
                        plsc.addupdate(bufs[slot].at[r, sl], pos_v[prow, sl])
                        return carry3

                    lax.fori_loop(0, D // LANES, add_j, 0, unroll=16)
                    return carry2

                lax.fori_loop(0, CH, row_fn, 0)
                scatter(u, j)
            return carry

        lax.fori_loop(0, NR, round_fn, 0)
        scatter_wait(T - 2, 2)
        scatter_wait(T - 1, 3)

    out = k(idx_r, token_table, pos_table)
    return out.reshape(B, S, D)


# speedup vs baseline: 2.1702x; 1.4584x over previous
import functools

import jax
import jax.numpy as jnp
from jax import lax
from jax.experimental import pallas as pl
from jax.experimental.pallas import tpu as pltpu
from jax.experimental.pallas import tpu_sc as plsc

LANES = 16


def kernel(input_ids, token_table, pos_table):
    B, S = input_ids.shape
    V, D = token_table.shape
    NW = 32
    CH = 8
    RW = (B * S) // NW
    T = RW // CH
    NR = T // 4

    idx_r = input_ids.astype(jnp.int32).reshape(NW, 1, RW)

    mesh = plsc.VectorSubcoreMesh(core_axis_name="c", subcore_axis_name="s")

    @functools.partial(
        pl.kernel,
        mesh=mesh,
        out_type=jax.ShapeDtypeStruct((B * S, D), jnp.float32),
        scratch_types=[
            pltpu.VMEM((S, D), jnp.float32),
            pltpu.VMEM((1, RW), jnp.int32),
            pltpu.VMEM((CH, D), jnp.float32),
            pltpu.VMEM((CH, D), jnp.float32),
            pltpu.VMEM((CH, D), jnp.float32),
            pltpu.VMEM((CH, D), jnp.float32),
            pltpu.SemaphoreType.DMA,
            pltpu.SemaphoreType.DMA,
            pltpu.SemaphoreType.DMA,
            pltpu.SemaphoreType.DMA,
            pltpu.SemaphoreType.DMA,
            pltpu.SemaphoreType.DMA,
            pltpu.SemaphoreType.DMA,
            pltpu.SemaphoreType.DMA,
        ],
    )
    def k(idx_hbm, tok_hbm, pos_hbm, out_hbm, pos_v, idx_v,
          b0, b1, b2, b3, g0, g1, g2, g3, s0, s1, s2, s3):
        bufs = [b0, b1, b2, b3]
        gsems = [g0, g1, g2, g3]
        ssems = [s0, s1, s2, s3]
        c = lax.axis_index("c")
        s = lax.axis_index("s")
        wid = s * 2 + c
        row0 = wid * RW
        pltpu.sync_copy(pos_hbm, pos_v)
        pltpu.sync_copy(idx_hbm.at[wid], idx_v)

        def gather(u, slot):
            pltpu.async_copy(
                tok_hbm.at[idx_v.at[0, pl.ds(u * CH, CH)]],
                bufs[slot], gsems[slot])

        def gather_wait(u, slot):
            pltpu.make_async_copy(
                tok_hbm.at[idx_v.at[0, pl.ds(u * CH, CH)]],
                bufs[slot], gsems[slot]).wait()

        def scatter(u, slot):
            pltpu.async_copy(
                bufs[slot], out_hbm.at[pl.ds(row0 + u * CH, CH), :],
                ssems[slot])

        def scatter_wait(u, slot):
            pltpu.make_async_copy(
                bufs[slot], out_hbm.at[pl.ds(row0 + u * CH, CH), :],
                ssems[slot]).wait()

        gather(0, 0)
        gather(1, 1)

        def round_fn(rd, carry):
            for j in range(4):
                u = rd * 4 + j
                pslot = (j - 2) % 4
                if j < 2:
                    @pl.when(rd > 0)
                    def _():
                        scatter_wait(u - 2, pslot)
                else:
                    scatter_wait(u - 2, pslot)
                if j < 2:
                    gather(u + 2, pslot)
                else:
                    @pl.when(rd < NR - 1)
                    def _():
                        gather(u + 2, pslot)
                gather_wait(u, j)
                base = lax.rem(row0 + u * CH, S)

                @plsc.parallel_loop(0, CH)
                def row_fn(r, slot=j):
                    prow = base + r
                    prow = lax.select(prow >= S, prow - S, prow)

                    @plsc.parallel_loop(0, D // LANES, unroll=16)
                    def add_j(jj):
                        sl = pl.ds(jj * LANES, LANES)
                        plsc.addupdate(bufs[slot].at[r, sl], pos_v[prow, sl])
                scatter(u, j)
            return carry

        lax.fori_loop(0, NR, round_fn, 0)
        scatter_wait(T - 2, 2)
        scatter_wait(T - 1, 3)

    out = k(idx_r, token_table, pos_table)
    return out.reshape(B, S, D)
